# Initial kernel scaffold; baseline (speedup 1.0000x reference)
import jax
import jax.numpy as jnp
from jax.experimental import pallas as pl


def kernel(ids, ids_table, pos_table):
    raise NotImplementedError("write your pallas kernel here")


# speedup vs baseline: 1.4942x; 1.4942x over previous
import functools

import jax
import jax.numpy as jnp
from jax import lax
from jax.experimental import pallas as pl
from jax.experimental.pallas import tpu as pltpu
from jax.experimental.pallas import tpu_sc as plsc


def _make_sc_embed(n_ids: int, vocab: int, d: int, seq: int):
  info = plsc.get_sparse_core_info()
  nc, ns = info.num_cores, info.num_subcores
  nw = nc * ns
  assert n_ids % nw == 0
  ids_per_w = n_ids // nw
  chunk = 32
  n_chunks = ids_per_w // chunk
  assert ids_per_w % chunk == 0
  assert seq % nw == 0
  pos_per_w = seq // nw
  n_pos_chunks = pos_per_w // chunk
  assert pos_per_w % chunk == 0

  mesh = plsc.VectorSubcoreMesh(core_axis_name="c", subcore_axis_name="s")

  @functools.partial(
      pl.kernel,
      mesh=mesh,
      out_type=(
          jax.ShapeDtypeStruct((n_ids, d), jnp.float32),
          jax.ShapeDtypeStruct((seq, d), jnp.float32),
      ),
      scratch_types=[
          pltpu.VMEM((ids_per_w,), jnp.int32),
          pltpu.VMEM((chunk, d), jnp.float32),
          pltpu.SemaphoreType.DMA,
      ],
  )
  def sc_embed(ids_hbm, table_hbm, pos_hbm, out_ids, out_pos, idx_v, buf, sem):
    wid = lax.axis_index("s") * nc + lax.axis_index("c")
    base = wid * ids_per_w
    pltpu.sync_copy(ids_hbm.at[pl.ds(base, ids_per_w)], idx_v)

    def gather_chunk(i, _):
      row0 = i * chunk
      pltpu.async_copy(
          table_hbm.at[idx_v.at[pl.ds(row0, chunk)]], buf, sem).wait()
      pltpu.sync_copy(buf, out_ids.at[pl.ds(base + row0, chunk)])
      return _

    lax.fori_loop(0, n_chunks, gather_chunk, 0)

    pbase = wid * pos_per_w

    def pos_chunk(i, _):
      row0 = pbase + i * chunk
      pltpu.sync_copy(pos_hbm.at[pl.ds(row0, chunk)], buf)
      pltpu.sync_copy(buf, out_pos.at[pl.ds(row0, chunk)])
      return _

    lax.fori_loop(0, n_pos_chunks, pos_chunk, 0)

  return sc_embed


def kernel(ids, ids_table, pos_table):
  b, s = ids.shape
  vocab, d = ids_table.shape
  ids_flat = ids.reshape(-1).astype(jnp.int32)
  sc_embed = _make_sc_embed(b * s, vocab, d, s)
  ids_emb, pos_emb = sc_embed(ids_flat, ids_table, pos_table)
  return ids_emb.reshape(b, s, d), pos_emb[None]
